# baseline (speedup 1.0000x reference)
import functools

import jax
import jax.numpy as jnp
from jax import lax
from jax.experimental import pallas as pl
from jax.experimental.pallas import tpu as pltpu
from jax.experimental.pallas import tpu_sc as plsc

EMB = 16
BATCH = 16384
VOCAB = 1_000_000
GROUPS_PER_DIM = VOCAB // 8

NUM_CORES = 2
NUM_SUBCORES = 16
NW = NUM_CORES * NUM_SUBCORES
BPW = BATCH // NW
IDXW = 128
NIDX = BPW // IDXW
CHUNK = 128
NCHUNK = BPW // CHUNK
CSPLIT = 8


def _sc_interac(x0, x1, f1, f2):
  mesh = plsc.VectorSubcoreMesh(core_axis_name="c", subcore_axis_name="s")

  @functools.partial(
      pl.kernel,
      mesh=mesh,
      out_type=jax.ShapeDtypeStruct((EMB, BATCH), jnp.float32),
      compiler_params=pltpu.CompilerParams(
          use_tc_tiling_on_sc=False, needs_layout_passes=False),
      scratch_types=[
          pltpu.VMEM((NIDX, IDXW), jnp.int32),
          pltpu.VMEM((NIDX, IDXW), jnp.int32),
          pltpu.VMEM((EMB, BPW), jnp.int32),
          pltpu.VMEM((EMB, BPW), jnp.int32),
          pltpu.VMEM((CSPLIT, BPW, 8), jnp.float32),
          pltpu.VMEM((CSPLIT, BPW, 8), jnp.float32),
          pltpu.VMEM((EMB, BPW), jnp.float32),
          pltpu.VMEM((EMB, BPW), jnp.float32),
          pltpu.SemaphoreType.DMA,
      ],
  )
  def k(x0_hbm, x1_hbm, f1_hbm, f2_hbm, out_hbm,
        idx1_v, idx2_v, grp1_v, grp2_v, raw1_v, raw2_v,
        cols1_v, cols2_v, sem):
    wid = lax.axis_index("s") * NUM_CORES + lax.axis_index("c")
    base = wid * BPW

    pltpu.sync_copy(x0_hbm.at[pl.ds(wid * NIDX, NIDX)], idx1_v)
    pltpu.sync_copy(x1_hbm.at[pl.ds(wid * NIDX, NIDX)], idx2_v)

    for c4 in range(NIDX):
      def expand(g, _, c4=c4):
        off = c4 * IDXW + g * EMB
        for iv, gv in ((idx1_v, grp1_v), (idx2_v, grp2_v)):
          h = iv[c4, pl.ds(g * EMB, EMB)] >> 3
          for c in range(EMB):
            gv[c, pl.ds(off, EMB)] = h + (c * GROUPS_PER_DIM)
        return 0
      lax.fori_loop(0, IDXW // EMB, expand, 0)

    for p in range(EMB // CSPLIT):
      def fire(j, _, p=p):
        c8 = j >> 2
        c = p * CSPLIT + c8
        off = (j & 3) * CHUNK
        pltpu.async_copy(f1_hbm.at[grp1_v.at[c, pl.ds(off, CHUNK)]],
                         raw1_v.at[c8, pl.ds(off, CHUNK)], sem)
        pltpu.async_copy(f2_hbm.at[grp2_v.at[c, pl.ds(off, CHUNK)]],
                         raw2_v.at[c8, pl.ds(off, CHUNK)], sem)
        return 0
      lax.fori_loop(0, CSPLIT * NCHUNK, fire, 0)

      def drain(j, _):
        pltpu.make_async_copy(f1_hbm.at[grp1_v.at[0, pl.ds(0, CHUNK)]],
                              raw1_v.at[0, pl.ds(0, CHUNK)], sem).wait()
        pltpu.make_async_copy(f2_hbm.at[grp2_v.at[0, pl.ds(0, CHUNK)]],
                              raw2_v.at[0, pl.ds(0, CHUNK)], sem).wait()
        return 0
      lax.fori_loop(0, CSPLIT * NCHUNK, drain, 0)

      def extract(j, _, p=p):
        c8 = j >> 5
        t = (j & 31) * EMB
        pos = t + lax.iota(jnp.int32, EMB)
        c8v = jnp.full((EMB,), c8, jnp.int32)
        for iv, rv, cv in ((idx1_v, raw1_v, cols1_v),
                           (idx2_v, raw2_v, cols2_v)):
          i = iv[t >> 7, pl.ds(t & 127, EMB)]
          vals = plsc.load_gather(rv, [c8v, pos, i & 7])
          cv[p * CSPLIT + c8, pl.ds(t, EMB)] = vals
        return 0
      lax.fori_loop(0, CSPLIT * (BPW // EMB), extract, 0)

    def mul(j, _):
      c = j >> 5
      t = (j & 31) * EMB
      cols1_v[c, pl.ds(t, EMB)] = (
          cols1_v[c, pl.ds(t, EMB)] * cols2_v[c, pl.ds(t, EMB)])
      return 0
    lax.fori_loop(0, EMB * (BPW // EMB), mul, 0)

    pltpu.sync_copy(cols1_v, out_hbm.at[:, pl.ds(base, BPW)])

  return k(x0, x1, f1, f2)


def kernel(x, emb1, emb2):
  x0 = x[0].reshape(NW * NIDX, IDXW)
  x1 = x[1].reshape(NW * NIDX, IDXW)
  f1 = emb1.T.reshape(2 * VOCAB, 8)
  f2 = emb2.T.reshape(2 * VOCAB, 8)
  out_t = _sc_interac(x0, x1, f1, f2)
  return out_t.T


# speedup vs baseline: 3.1789x; 3.1789x over previous
import functools

import jax
import jax.numpy as jnp
from jax import lax
from jax.experimental import pallas as pl
from jax.experimental.pallas import tpu as pltpu
from jax.experimental.pallas import tpu_sc as plsc

EMB = 16
BATCH = 16384
NUM_CORES = 2
NUM_SUBCORES = 16
NW = NUM_CORES * NUM_SUBCORES
BPW = BATCH // NW
IDXW = 128
NIDX = BPW // IDXW
CHUNK = 128
NCHUNK = BPW // CHUNK


def _sc_interac(x0, x1, e1, e2):
  mesh = plsc.VectorSubcoreMesh(core_axis_name="c", subcore_axis_name="s")

  @functools.partial(
      pl.kernel,
      mesh=mesh,
      out_type=jax.ShapeDtypeStruct((EMB, BATCH), jnp.float32),
      compiler_params=pltpu.CompilerParams(
          use_tc_tiling_on_sc=False, needs_layout_passes=False),
      scratch_types=[
          pltpu.VMEM((NIDX, IDXW), jnp.int32),
          pltpu.VMEM((NIDX, IDXW), jnp.int32),
          pltpu.VMEM((BPW, EMB), jnp.float32),
          pltpu.VMEM((BPW, EMB), jnp.float32),
          pltpu.VMEM((EMB, BPW), jnp.float32),
          pltpu.SemaphoreType.DMA,
      ],
  )
  def k(x0_hbm, x1_hbm, e1_hbm, e2_hbm, out_hbm,
        idx1_v, idx2_v, rows1_v, rows2_v, cols_v, sem):
    wid = lax.axis_index("s") * NUM_CORES + lax.axis_index("c")
    base = wid * BPW

    pltpu.sync_copy(x0_hbm.at[pl.ds(wid * NIDX, NIDX)], idx1_v)
    pltpu.sync_copy(x1_hbm.at[pl.ds(wid * NIDX, NIDX)], idx2_v)

    copies = []
    for c in range(NCHUNK):
      copies.append(pltpu.async_copy(
          e1_hbm.at[idx1_v.at[c]],
          rows1_v.at[pl.ds(c * CHUNK, CHUNK)], sem))
      copies.append(pltpu.async_copy(
          e2_hbm.at[idx2_v.at[c]],
          rows2_v.at[pl.ds(c * CHUNK, CHUNK)], sem))
    for cp in copies:
      cp.wait()

    def mul(r, _):
      rows1_v[r, :] = rows1_v[r, :] * rows2_v[r, :]
      return 0
    lax.fori_loop(0, BPW, mul, 0)

    def transp(j, _):
      c = j >> 5
      t = (j & 31) * EMB
      pos = t + lax.iota(jnp.int32, EMB)
      cv = jnp.full((EMB,), c, jnp.int32)
      cols_v[c, pl.ds(t, EMB)] = plsc.load_gather(rows1_v, [pos, cv])
      return 0
    lax.fori_loop(0, EMB * (BPW // EMB), transp, 0)

    pltpu.sync_copy(cols_v, out_hbm.at[:, pl.ds(base, BPW)])

  return k(x0, x1, e1, e2)


def kernel(x, emb1, emb2):
  x0 = x[0].reshape(NW * NIDX, IDXW)
  x1 = x[1].reshape(NW * NIDX, IDXW)
  out_t = _sc_interac(x0, x1, emb1, emb2)
  return out_t.T


# speedup vs baseline: 4.6917x; 1.4759x over previous
import functools

import jax
import jax.numpy as jnp
from jax import lax
from jax.experimental import pallas as pl
from jax.experimental.pallas import tpu as pltpu
from jax.experimental.pallas import tpu_sc as plsc

EMB = 16
BATCH = 16384
NUM_CORES = 2
NUM_SUBCORES = 16
NW = NUM_CORES * NUM_SUBCORES
BPW = BATCH // NW
HALF = BPW // 2
IDXW = 128
NIDX = BPW // IDXW


def _sc_interac(x0, x1, emb1, emb2):
  mesh = plsc.VectorSubcoreMesh(core_axis_name="c", subcore_axis_name="s")

  @functools.partial(
      pl.kernel,
      mesh=mesh,
      out_type=jax.ShapeDtypeStruct((BATCH, EMB), jnp.float32),
      scratch_types=[
          pltpu.VMEM((NIDX, IDXW), jnp.int32),
          pltpu.VMEM((NIDX, IDXW), jnp.int32),
          pltpu.VMEM((HALF, EMB), jnp.float32),
          pltpu.VMEM((HALF, EMB), jnp.float32),
          pltpu.SemaphoreType.DMA,
      ],
  )
  def k(x0_hbm, x1_hbm, e1_hbm, e2_hbm, out_hbm,
        idx1_v, idx2_v, rows1_v, rows2_v, sem):
    wid = lax.axis_index("s") * NUM_CORES + lax.axis_index("c")
    base = wid * BPW

    pltpu.sync_copy(x0_hbm.at[pl.ds(wid * NIDX, NIDX)], idx1_v)
    pltpu.sync_copy(x1_hbm.at[pl.ds(wid * NIDX, NIDX)], idx2_v)

    for h in range(2):
      for c in range(2 * h, 2 * h + 2):
        def fire(g, _, c=c):
          iv1 = idx1_v[c, pl.ds(g * EMB, EMB)]
          iv2 = idx2_v[c, pl.ds(g * EMB, EMB)]
          rbase = (c - 2 * h) * IDXW + g * EMB
          for l in range(EMB):
            pltpu.async_copy(e1_hbm.at[pl.ds(iv1[l], 1)],
                             rows1_v.at[pl.ds(rbase + l, 1)], sem)
            pltpu.async_copy(e2_hbm.at[pl.ds(iv2[l], 1)],
                             rows2_v.at[pl.ds(rbase + l, 1)], sem)
          return 0
        lax.fori_loop(0, IDXW // EMB, fire, 0)

      def drain(r, _):
        pltpu.make_async_copy(
            e1_hbm.at[pl.ds(0, 1)], rows1_v.at[pl.ds(0, 1)], sem).wait()
        pltpu.make_async_copy(
            e2_hbm.at[pl.ds(0, 1)], rows2_v.at[pl.ds(0, 1)], sem).wait()
        return 0
      lax.fori_loop(0, HALF, drain, 0)

      def mul(r, _):
        rows1_v[r, :] = rows1_v[r, :] * rows2_v[r, :]
        return 0
      lax.fori_loop(0, HALF, mul, 0)

      pltpu.sync_copy(rows1_v, out_hbm.at[pl.ds(base + h * HALF, HALF)])

  return k(x0, x1, emb1, emb2)


def kernel(x, emb1, emb2):
  x0 = x[0].reshape(NW * NIDX, IDXW)
  x1 = x[1].reshape(NW * NIDX, IDXW)
  return _sc_interac(x0, x1, emb1, emb2)
